# Initial kernel scaffold; baseline (speedup 1.0000x reference)
import jax
import jax.numpy as jnp
from jax.experimental import pallas as pl


def kernel(inputs, tables, W, b):
    raise NotImplementedError("write your pallas kernel here")


# speedup vs baseline: 6.4427x; 6.4427x over previous
import functools

import jax
import jax.numpy as jnp
from jax import lax
from jax.experimental import pallas as pl
from jax.experimental.pallas import tpu as pltpu
from jax.experimental.pallas import tpu_sc as plsc

NUM_TABLES = 26
VOCAB = 100000
EMBED_DIM = 32
BATCH = 16384
DENSE_OUT = 5
CONCAT = NUM_TABLES * EMBED_DIM

_NC = 2
_NS = 16
_NW = _NC * _NS
_BPW = BATCH // _NW
_RB = 128
_NBLK = _BPW // _RB
_IDXB = NUM_TABLES * _RB
_GROW = 128
_L = 16


def _make_sc_gather():
    mesh = plsc.VectorSubcoreMesh(core_axis_name="c", subcore_axis_name="s")

    @functools.partial(
        pl.kernel,
        mesh=mesh,
        compiler_params=pltpu.CompilerParams(needs_layout_passes=False),
        out_type=jax.ShapeDtypeStruct((NUM_TABLES, BATCH, EMBED_DIM), jnp.float32),
        scratch_types=[
            pltpu.VMEM((_IDXB,), jnp.int32),
            pltpu.VMEM((_IDXB,), jnp.int32),
            pltpu.VMEM((_RB, _GROW), jnp.float32),
            pltpu.VMEM((_RB, EMBED_DIM), jnp.float32),
            pltpu.SemaphoreType.DMA,
        ],
    )
    def gather_k(grp_hbm, cb_hbm, tab_hbm, out_hbm, grp_v, cb_v, rows_v, sel_v, sem):
        wid = lax.axis_index("s") * _NC + lax.axis_index("c")
        base = wid * _BPW
        iota = lax.iota(jnp.int32, _L)

        def blk_body(r, carry):
            row0 = base + r * _RB
            blk = wid * _NBLK + r
            pltpu.sync_copy(grp_hbm.at[pl.ds(blk * _IDXB, _IDXB)], grp_v)
            pltpu.sync_copy(cb_hbm.at[pl.ds(blk * _IDXB, _IDXB)], cb_v)
            for t in range(NUM_TABLES):
                pltpu.async_copy(
                    tab_hbm.at[grp_v.at[pl.ds(t * _RB, _RB)]], rows_v, sem
                ).wait()

                def sel_body(rr, c, t=t):
                    splat_rr = jnp.full((_L,), rr, jnp.int32)
                    cb = plsc.load_gather(cb_v, [splat_rr + (t * _RB)])
                    c0 = cb + iota
                    v0 = plsc.load_gather(rows_v, [splat_rr, c0])
                    v1 = plsc.load_gather(rows_v, [splat_rr, c0 + _L])
                    plsc.store_scatter(sel_v, [splat_rr, iota], v0)
                    plsc.store_scatter(sel_v, [splat_rr, iota + _L], v1)
                    return c

                lax.fori_loop(0, _RB, sel_body, 0)
                pltpu.sync_copy(sel_v, out_hbm.at[t, pl.ds(row0, _RB), :])
            return carry

        lax.fori_loop(0, _NBLK, blk_body, 0)

    return gather_k


_sc_gather = _make_sc_gather()


def _tc_dense(x3, w3, bias2d):
    bm = 2048

    def mm_k(x_ref, w_ref, b_ref, o_ref):
        acc = jnp.broadcast_to(b_ref[...], (bm, DENSE_OUT))
        for t in range(NUM_TABLES):
            acc = acc + jnp.dot(
                x_ref[t], w_ref[t], preferred_element_type=jnp.float32
            )
        o_ref[...] = acc

    return pl.pallas_call(
        mm_k,
        grid=(BATCH // bm,),
        in_specs=[
            pl.BlockSpec((NUM_TABLES, bm, EMBED_DIM), lambda i: (0, i, 0)),
            pl.BlockSpec((NUM_TABLES, EMBED_DIM, DENSE_OUT), lambda i: (0, 0, 0)),
            pl.BlockSpec((1, DENSE_OUT), lambda i: (0, 0)),
        ],
        out_specs=pl.BlockSpec((bm, DENSE_OUT), lambda i: (i, 0)),
        out_shape=jax.ShapeDtypeStruct((BATCH, DENSE_OUT), jnp.float32),
    )(x3, w3, bias2d)


def _blockify(a):
    return jnp.transpose(
        a.reshape(NUM_TABLES, BATCH // _RB, _RB), (1, 0, 2)
    ).reshape(-1)


def kernel(inputs, tables, W, b):
    offs = (jnp.arange(NUM_TABLES, dtype=jnp.int32) * VOCAB)[:, None]
    idxf = inputs + offs
    grp = _blockify(idxf >> 2)
    cbase = _blockify((idxf & 3) << 5)
    tab4 = tables.reshape(NUM_TABLES * VOCAB // 4, _GROW)
    x3 = _sc_gather(grp, cbase, tab4)
    w3 = W.reshape(NUM_TABLES, EMBED_DIM, DENSE_OUT)
    return _tc_dense(x3, w3, b.reshape(1, DENSE_OUT))
